# baseline (speedup 1.0000x reference)
import functools
import math

import jax
import jax.numpy as jnp
from jax import lax
from jax.experimental import pallas as pl
from jax.experimental.pallas import tpu as pltpu
from jax.experimental.pallas import tpu_sc as plsc

EMB_DIM = 164
SEQ_LEN = 20
SCALE = math.sqrt(float(EMB_DIM))
LANES = 16

NUM_CORES = 2
NUM_SUBCORES = 16
NUM_WORKERS = NUM_CORES * NUM_SUBCORES


def _pe_table():
    position = jnp.arange(0, SEQ_LEN, dtype=jnp.float32)[:, None]
    div_term = jnp.exp(
        jnp.arange(0, EMB_DIM, 2, dtype=jnp.float32) * -(math.log(10000.0) / EMB_DIM)
    )
    angles = position * div_term
    pe = jnp.zeros((SEQ_LEN, EMB_DIM), dtype=jnp.float32)
    pe = pe.at[:, 0::2].set(jnp.sin(angles))
    pe = pe.at[:, 1::2].set(jnp.cos(angles))
    return pe


def kernel(input_ids, embedding_weight):
    n_seq, seq_len = input_ids.shape
    spw = n_seq // NUM_WORKERS

    ids = input_ids.astype(jnp.int32)
    pe = _pe_table()

    mesh = plsc.VectorSubcoreMesh(core_axis_name="c", subcore_axis_name="s")

    DBLOCKS = [k * LANES for k in range(EMB_DIM // LANES)] + [EMB_DIM - LANES]

    @functools.partial(
        pl.kernel,
        mesh=mesh,
        out_type=jax.ShapeDtypeStruct((SEQ_LEN, EMB_DIM, n_seq), jnp.float32),
        compiler_params=pltpu.CompilerParams(use_tc_tiling_on_sc=True, needs_layout_passes=False),
        scratch_types=[
            pltpu.VMEM((spw, SEQ_LEN), jnp.int32),
            pltpu.VMEM((SEQ_LEN, spw), jnp.int32),
            pltpu.VMEM((SEQ_LEN, EMB_DIM), jnp.float32),
            pltpu.VMEM((spw, EMB_DIM), jnp.float32),
            pltpu.VMEM((spw, EMB_DIM), jnp.float32),
            pltpu.VMEM((EMB_DIM, spw), jnp.float32),
            pltpu.SemaphoreType.DMA,
            pltpu.SemaphoreType.DMA,
        ],
    )
    def _emb(ids_hbm, table_hbm, pe_hbm, out_hbm, idx_v, idx_t, pe_v, buf0, buf1, tbuf, s0, s1):
        wid = lax.axis_index("s") * NUM_CORES + lax.axis_index("c")
        seq_base = wid * spw
        pltpu.sync_copy(ids_hbm.at[pl.ds(seq_base, spw)], idx_v)
        pltpu.sync_copy(pe_hbm, pe_v)

        for p in range(SEQ_LEN):
            for g in range(spw // LANES):
                row_idx = g * LANES + lax.iota(jnp.int32, LANES)
                col_idx = jnp.full((LANES,), p, jnp.int32)
                idx_t[p, pl.ds(g * LANES, LANES)] = plsc.load_gather(
                    idx_v, [row_idx, col_idx]
                )

        bufs = (buf0, buf1)
        sems = (s0, s1)

        def issue_gather(p, buf, sem):
            for g in range(spw // LANES):
                vec = idx_t[p, pl.ds(g * LANES, LANES)]
                for l in range(LANES):
                    pltpu.async_copy(
                        table_hbm.at[vec[l]], buf.at[g * LANES + l], sem
                    )

        def wait_gather(buf, sem):
            for s in range(spw):
                pltpu.make_async_copy(table_hbm.at[0], buf.at[s], sem).wait()

        def compute_transpose(p, buf):
            def grp_body(g, carry):
                n0 = g * LANES
                row_idx = n0 + lax.iota(jnp.int32, LANES)
                for d0 in DBLOCKS:
                    pe_vec = pe_v[p, pl.ds(d0, LANES)]
                    for dl in range(LANES):
                        col_idx = jnp.full((LANES,), d0 + dl, jnp.int32)
                        g16 = plsc.load_gather(buf, [row_idx, col_idx])
                        val = g16 * SCALE + lax.broadcast(pe_vec[dl], (LANES,))
                        tbuf[d0 + dl, pl.ds(n0, LANES)] = val
                return carry

            lax.fori_loop(0, spw // LANES, grp_body, 0)

        issue_gather(0, buf0, s0)

        def loop_body(p0, carry):
            for b in range(2):
                p = p0 + b
                nxt = p + 1

                @pl.when(nxt < SEQ_LEN)
                def _():
                    issue_gather(nxt, bufs[1 - b], sems[1 - b])

                wait_gather(bufs[b], sems[b])
                compute_transpose(p, bufs[b])
                pltpu.sync_copy(
                    tbuf, out_hbm.at[p, :, pl.ds(seq_base, spw)]
                )
            return carry

        lax.fori_loop(0, SEQ_LEN // 2, lambda i, cr: loop_body(i * 2, cr), 0)

    out = _emb(ids, embedding_weight, pe)
    return jnp.transpose(out, (2, 0, 1))


# speedup vs baseline: 1.2784x; 1.2784x over previous
import functools
import math

import jax
import jax.numpy as jnp
from jax import lax
from jax.experimental import pallas as pl
from jax.experimental.pallas import tpu as pltpu
from jax.experimental.pallas import tpu_sc as plsc

EMB_DIM = 164
SEQ_LEN = 20
SCALE = math.sqrt(float(EMB_DIM))
LANES = 16

NUM_CORES = 2
NUM_SUBCORES = 16
NUM_WORKERS = NUM_CORES * NUM_SUBCORES

SEQS_PER_CHUNK = 8
ROWS_PER_CHUNK = SEQS_PER_CHUNK * SEQ_LEN


def _pe_table():
    position = jnp.arange(0, SEQ_LEN, dtype=jnp.float32)[:, None]
    div_term = jnp.exp(
        jnp.arange(0, EMB_DIM, 2, dtype=jnp.float32) * -(math.log(10000.0) / EMB_DIM)
    )
    angles = position * div_term
    pe = jnp.zeros((SEQ_LEN, EMB_DIM), dtype=jnp.float32)
    pe = pe.at[:, 0::2].set(jnp.sin(angles))
    pe = pe.at[:, 1::2].set(jnp.cos(angles))
    return pe


def kernel(input_ids, embedding_weight):
    n_seq, seq_len = input_ids.shape
    seqs_per_worker = n_seq // NUM_WORKERS
    n_chunks = seqs_per_worker // SEQS_PER_CHUNK

    ids = input_ids.astype(jnp.int32)
    pe = _pe_table()

    mesh = plsc.VectorSubcoreMesh(core_axis_name="c", subcore_axis_name="s")

    @functools.partial(
        pl.kernel,
        mesh=mesh,
        out_type=jax.ShapeDtypeStruct((n_seq, SEQ_LEN, EMB_DIM), jnp.float32),
        compiler_params=pltpu.CompilerParams(use_tc_tiling_on_sc=True),
        scratch_types=[
            pltpu.VMEM((seqs_per_worker, SEQ_LEN), jnp.int32),
            pltpu.VMEM((SEQ_LEN, EMB_DIM), jnp.float32),
            pltpu.VMEM((SEQS_PER_CHUNK, SEQ_LEN, EMB_DIM), jnp.float32),
            pltpu.VMEM((SEQS_PER_CHUNK, SEQ_LEN, EMB_DIM), jnp.float32),
            pltpu.SemaphoreType.DMA,
            pltpu.SemaphoreType.DMA,
        ],
    )
    def _emb(ids_hbm, table_hbm, pe_hbm, out_hbm, idx_v, pe_v, buf0, buf1, s0, s1):
        wid = lax.axis_index("s") * NUM_CORES + lax.axis_index("c")
        seq_base = wid * seqs_per_worker
        pltpu.sync_copy(ids_hbm.at[pl.ds(seq_base, seqs_per_worker)], idx_v)
        pltpu.sync_copy(pe_hbm, pe_v)

        bufs = (buf0, buf1)
        sems = (s0, s1)

        def issue_gather(c, buf, sem):
            for q in range(SEQS_PER_CHUNK):
                sq = c * SEQS_PER_CHUNK + q
                lo = idx_v[sq, pl.ds(0, LANES)]
                hi = idx_v[sq, pl.ds(SEQ_LEN - LANES, LANES)]
                for p in range(SEQ_LEN):
                    rid = lo[p] if p < LANES else hi[p - (SEQ_LEN - LANES)]
                    pltpu.async_copy(table_hbm.at[rid], buf.at[q, p], sem)

        def wait_gather(buf, sem):
            for q in range(SEQS_PER_CHUNK):
                for p in range(SEQ_LEN):
                    pltpu.make_async_copy(
                        table_hbm.at[0], buf.at[q, p], sem
                    ).wait()

        def compute(buf):
            def seq_body(q, carry):
                for p in range(SEQ_LEN):
                    slices = [
                        pl.ds(k * LANES, LANES) for k in range(EMB_DIM // LANES)
                    ] + [pl.ds(EMB_DIM - LANES, LANES)]
                    vals = [buf[q, p, sl] * SCALE + pe_v[p, sl] for sl in slices]
                    for sl, v in zip(slices, vals):
                        buf[q, p, sl] = v
                return carry

            lax.fori_loop(0, SEQS_PER_CHUNK, seq_body, 0)

        issue_gather(0, buf0, s0)

        def loop_body(c0, carry):
            for b in range(2):
                c = c0 + b
                nxt = c + 1

                @pl.when(nxt < n_chunks)
                def _():
                    issue_gather(nxt, bufs[1 - b], sems[1 - b])

                wait_gather(bufs[b], sems[b])
                compute(bufs[b])
                pltpu.sync_copy(
                    bufs[b],
                    out_hbm.at[pl.ds(seq_base + c * SEQS_PER_CHUNK, SEQS_PER_CHUNK)],
                )
            return carry

        lax.fori_loop(0, n_chunks // 2, lambda i, cr: loop_body(i * 2, cr), 0)

    return _emb(ids, embedding_weight, pe)
